# baseline (speedup 1.0000x reference)
import functools

import jax
import jax.numpy as jnp
from jax import lax
from jax.experimental import pallas as pl
from jax.experimental.pallas import tpu as pltpu
from jax.experimental.pallas import tpu_sc as plsc

N = 50000
E = 1600000
NPAD = 50176
S = NPAD // 32
EPT = E // 32
C = 2000
NCH = EPT // C
NV = C // 16
DAMP = 0.85
MAXIT = 100


def _make_kernel():
    mesh = plsc.VectorSubcoreMesh(core_axis_name="c", subcore_axis_name="s")

    @functools.partial(
        pl.kernel,
        out_type=(
            jax.ShapeDtypeStruct((NPAD,), jnp.float32),
            jax.ShapeDtypeStruct((NPAD,), jnp.float32),
        ),
        mesh=mesh,
        scratch_types=[
            pltpu.VMEM((NPAD,), jnp.float32),
            pltpu.VMEM((NPAD,), jnp.float32),
            pltpu.VMEM((2, C), jnp.int32),
            pltpu.VMEM((2, C), jnp.int32),
            pltpu.VMEM((2, C), jnp.float32),
            pltpu.HBM((32, NPAD), jnp.float32),
            pltpu.HBM((NPAD,), jnp.float32),
            pltpu.HBM((E,), jnp.int32),
            pltpu.HBM((E,), jnp.float32),
            pltpu.SemaphoreType.DMA,
            pltpu.SemaphoreType.DMA,
            pltpu.SemaphoreType.DMA,
            pltpu.SemaphoreType.REGULAR,
        ],
        compiler_params=pltpu.CompilerParams(needs_layout_passes=False, use_tc_tiling_on_sc=False),
    )
    def k(row_hbm, col_hbm, w_hbm, deg_out, pr_out,
          pr_v, acc_v, b0, b1, b2,
          partial_hbm, prb_hbm, pk_hbm, wn_hbm,
          sem0, sem1, sem2, bsem):
        cid = lax.axis_index("c")
        sid = lax.axis_index("s")
        wid = sid * 2 + cid
        ebase = wid * EPT
        sbase = wid * S

        def full_barrier():
            plsc.subcore_barrier()
            pltpu.core_barrier(bsem, core_axis_name="c")

        def zero_vmem(ref):
            z = jnp.zeros((16,), jnp.float32)

            @plsc.parallel_loop(0, NPAD // 16, unroll=8)
            def _(j):
                ref[pl.ds(j * 16, 16)] = z

        def reduce_slices_from_acc(damped):
            @plsc.parallel_loop(0, S // 16, unroll=2)
            def _(j):
                s = acc_v[pl.ds(j * 16, 16)]
                for t in range(1, 32):
                    s = s + acc_v[pl.ds(t * S + j * 16, 16)]
                if damped:
                    s = (1.0 - DAMP) / N + DAMP * s
                acc_v[pl.ds(j * 16, 16)] = s

        def stage_partials():
            descs = []
            for t in range(32):
                descs.append(pltpu.async_copy(
                    partial_hbm.at[t, pl.ds(sbase, S)],
                    acc_v.at[pl.ds(t * S, S)], sem2))
            for d in descs:
                d.wait()

        zero_vmem(acc_v)
        zero_vmem(pr_v)
        ones16 = jnp.full((16,), 1.0, jnp.float32)

        for c in range(NCH):
            ph = c & 1
            if c == 0:
                pltpu.async_copy(row_hbm.at[pl.ds(ebase, C)], b0.at[0], sem0).wait()
                pltpu.async_copy(w_hbm.at[pl.ds(ebase, C)], b2.at[0], sem1).wait()
            if c + 1 < NCH:
                nb = ebase + (c + 1) * C
                nph = (c + 1) & 1
                d_r = pltpu.async_copy(row_hbm.at[pl.ds(nb, C)], b0.at[nph], sem0)
                d_w = pltpu.async_copy(w_hbm.at[pl.ds(nb, C)], b2.at[nph], sem1)

            @plsc.parallel_loop(0, NV, unroll=8)
            def _(j, ph=ph):
                row = b0[ph, pl.ds(j * 16, 16)]
                w = b2[ph, pl.ds(j * 16, 16)]
                plsc.addupdate_scatter(acc_v, [row], ones16)
                plsc.addupdate_scatter(pr_v, [row], w)
            if c + 1 < NCH:
                d_r.wait()
                d_w.wait()

        pltpu.sync_copy(acc_v, partial_hbm.at[wid])
        full_barrier()
        stage_partials()
        reduce_slices_from_acc(damped=False)
        pltpu.sync_copy(acc_v.at[pl.ds(0, S)], deg_out.at[pl.ds(sbase, S)])

        pltpu.sync_copy(pr_v, partial_hbm.at[wid])
        full_barrier()
        stage_partials()
        reduce_slices_from_acc(damped=False)
        pltpu.sync_copy(acc_v.at[pl.ds(0, S)], prb_hbm.at[pl.ds(sbase, S)])
        full_barrier()
        pltpu.sync_copy(prb_hbm, pr_v)

        for c in range(NCH):
            base = ebase + c * C
            d_r = pltpu.async_copy(row_hbm.at[pl.ds(base, C)], b0.at[0], sem0)
            d_c = pltpu.async_copy(col_hbm.at[pl.ds(base, C)], b1.at[0], sem1)
            d_w = pltpu.async_copy(w_hbm.at[pl.ds(base, C)], b2.at[0], sem2)
            d_r.wait()
            d_c.wait()
            d_w.wait()

            @plsc.parallel_loop(0, NV, unroll=4)
            def _(j):
                row = b0[0, pl.ds(j * 16, 16)]
                col = b1[0, pl.ds(j * 16, 16)]
                w = b2[0, pl.ds(j * 16, 16)]
                od = plsc.load_gather(pr_v, [row])
                wn = w / (od + 1e-8)
                pk = lax.shift_left(col, 16) | row
                b1[0, pl.ds(j * 16, 16)] = pk
                b2[0, pl.ds(j * 16, 16)] = wn
            pltpu.sync_copy(b1.at[0], pk_hbm.at[pl.ds(base, C)])
            pltpu.sync_copy(b2.at[0], wn_hbm.at[pl.ds(base, C)])

        inv = jnp.full((16,), 1.0 / N, jnp.float32)

        @plsc.parallel_loop(0, NPAD // 16, unroll=8)
        def _(j):
            pr_v[pl.ds(j * 16, 16)] = inv

        def iteration(i, _):
            zero_vmem(acc_v)
            for c in range(NCH):
                ph = c & 1
                if c == 0:
                    pltpu.async_copy(pk_hbm.at[pl.ds(ebase, C)], b0.at[0], sem0).wait()
                    pltpu.async_copy(wn_hbm.at[pl.ds(ebase, C)], b2.at[0], sem1).wait()
                if c + 1 < NCH:
                    nb = ebase + (c + 1) * C
                    nph = (c + 1) & 1
                    d_p = pltpu.async_copy(pk_hbm.at[pl.ds(nb, C)], b0.at[nph], sem0)
                    d_n = pltpu.async_copy(wn_hbm.at[pl.ds(nb, C)], b2.at[nph], sem1)

                @plsc.parallel_loop(0, NV, unroll=8)
                def _(j, ph=ph):
                    pk = b0[ph, pl.ds(j * 16, 16)]
                    wn = b2[ph, pl.ds(j * 16, 16)]
                    row = pk & 0xFFFF
                    col = lax.shift_right_logical(pk, 16)
                    prg = plsc.load_gather(pr_v, [row])
                    plsc.addupdate_scatter(acc_v, [col], prg * wn)
                if c + 1 < NCH:
                    d_p.wait()
                    d_n.wait()

            pltpu.sync_copy(acc_v, partial_hbm.at[wid])
            full_barrier()
            stage_partials()
            reduce_slices_from_acc(damped=True)
            pltpu.sync_copy(acc_v.at[pl.ds(0, S)], prb_hbm.at[pl.ds(sbase, S)])
            full_barrier()
            pltpu.sync_copy(prb_hbm, pr_v)
            return 0

        lax.fori_loop(0, MAXIT, iteration, 0)

        pltpu.sync_copy(pr_v.at[pl.ds(sbase, S)], pr_out.at[pl.ds(sbase, S)])

    return k


_sc_kernel = _make_kernel()


def kernel(edge_index, num_nodes, edge_weight):
    row = edge_index[0]
    col = edge_index[1]
    deg, pr = _sc_kernel(row, col, edge_weight)
    embedding_dim = 16
    pad = jnp.zeros((N, embedding_dim - 2), dtype=jnp.float32)
    return jnp.concatenate([deg[:N, None], pr[:N, None], pad], axis=1)


# speedup vs baseline: 1.0254x; 1.0254x over previous
import functools

import jax
import jax.numpy as jnp
from jax import lax
from jax.experimental import pallas as pl
from jax.experimental.pallas import tpu as pltpu
from jax.experimental.pallas import tpu_sc as plsc

N = 50000
E = 1600000
NPAD = 50176
S = NPAD // 32
EPT = E // 32
C = 2000
NCH = EPT // C
NV = C // 16
DAMP = 0.85
MAXIT = 100


def _make_kernel():
    mesh = plsc.VectorSubcoreMesh(core_axis_name="c", subcore_axis_name="s")

    @functools.partial(
        pl.kernel,
        out_type=(
            jax.ShapeDtypeStruct((NPAD,), jnp.float32),
            jax.ShapeDtypeStruct((NPAD,), jnp.float32),
        ),
        mesh=mesh,
        scratch_types=[
            pltpu.VMEM((NPAD,), jnp.float32),
            pltpu.VMEM((NPAD,), jnp.float32),
            pltpu.VMEM((2, C), jnp.int32),
            pltpu.VMEM((2, C), jnp.int32),
            pltpu.VMEM((2, C), jnp.float32),
            pltpu.HBM((32, NPAD), jnp.float32),
            pltpu.HBM((NPAD,), jnp.float32),
            pltpu.HBM((E,), jnp.int32),
            pltpu.HBM((E,), jnp.float32),
            pltpu.SemaphoreType.DMA,
            pltpu.SemaphoreType.DMA,
            pltpu.SemaphoreType.DMA,
            pltpu.SemaphoreType.REGULAR,
        ],
        compiler_params=pltpu.CompilerParams(needs_layout_passes=False, use_tc_tiling_on_sc=False),
    )
    def k(row_hbm, col_hbm, w_hbm, deg_out, pr_out,
          pr_v, acc_v, b0, b1, b2,
          partial_hbm, prb_hbm, pk_hbm, wn_hbm,
          sem0, sem1, sem2, bsem):
        cid = lax.axis_index("c")
        sid = lax.axis_index("s")
        wid = sid * 2 + cid
        ebase = wid * EPT
        sbase = wid * S

        def full_barrier():
            plsc.subcore_barrier()
            pltpu.core_barrier(bsem, core_axis_name="c")

        def zero_vmem(ref):
            z = jnp.zeros((16,), jnp.float32)

            @plsc.parallel_loop(0, NPAD // 16, unroll=8)
            def _(j):
                ref[pl.ds(j * 16, 16)] = z

        def reduce_slices_from_acc(damped):
            @plsc.parallel_loop(0, S // 16, unroll=2)
            def _(j):
                s = acc_v[pl.ds(j * 16, 16)]
                for t in range(1, 32):
                    s = s + acc_v[pl.ds(t * S + j * 16, 16)]
                if damped:
                    s = (1.0 - DAMP) / N + DAMP * s
                acc_v[pl.ds(j * 16, 16)] = s

        def publish_partials(src_ref):
            descs = []
            for t in range(32):
                descs.append(pltpu.async_copy(
                    src_ref.at[pl.ds(t * S, S)],
                    partial_hbm.at[t, pl.ds(wid * S, S)], sem2))
            for d in descs:
                d.wait()

        def stage_partials():
            pltpu.async_copy(partial_hbm.at[wid], acc_v, sem2).wait()

        zero_vmem(acc_v)
        zero_vmem(pr_v)
        ones16 = jnp.full((16,), 1.0, jnp.float32)

        for c in range(NCH):
            ph = c & 1
            if c == 0:
                pltpu.async_copy(row_hbm.at[pl.ds(ebase, C)], b0.at[0], sem0).wait()
                pltpu.async_copy(w_hbm.at[pl.ds(ebase, C)], b2.at[0], sem1).wait()
            if c + 1 < NCH:
                nb = ebase + (c + 1) * C
                nph = (c + 1) & 1
                d_r = pltpu.async_copy(row_hbm.at[pl.ds(nb, C)], b0.at[nph], sem0)
                d_w = pltpu.async_copy(w_hbm.at[pl.ds(nb, C)], b2.at[nph], sem1)

            @plsc.parallel_loop(0, NV, unroll=8)
            def _(j, ph=ph):
                row = b0[ph, pl.ds(j * 16, 16)]
                w = b2[ph, pl.ds(j * 16, 16)]
                plsc.addupdate_scatter(acc_v, [row], ones16)
                plsc.addupdate_scatter(pr_v, [row], w)
            if c + 1 < NCH:
                d_r.wait()
                d_w.wait()

        publish_partials(acc_v)
        full_barrier()
        stage_partials()
        reduce_slices_from_acc(damped=False)
        pltpu.sync_copy(acc_v.at[pl.ds(0, S)], deg_out.at[pl.ds(sbase, S)])

        publish_partials(pr_v)
        full_barrier()
        stage_partials()
        reduce_slices_from_acc(damped=False)
        pltpu.sync_copy(acc_v.at[pl.ds(0, S)], prb_hbm.at[pl.ds(sbase, S)])
        full_barrier()
        pltpu.sync_copy(prb_hbm, pr_v)

        for c in range(NCH):
            base = ebase + c * C
            d_r = pltpu.async_copy(row_hbm.at[pl.ds(base, C)], b0.at[0], sem0)
            d_c = pltpu.async_copy(col_hbm.at[pl.ds(base, C)], b1.at[0], sem1)
            d_w = pltpu.async_copy(w_hbm.at[pl.ds(base, C)], b2.at[0], sem2)
            d_r.wait()
            d_c.wait()
            d_w.wait()

            @plsc.parallel_loop(0, NV, unroll=4)
            def _(j):
                row = b0[0, pl.ds(j * 16, 16)]
                col = b1[0, pl.ds(j * 16, 16)]
                w = b2[0, pl.ds(j * 16, 16)]
                od = plsc.load_gather(pr_v, [row])
                wn = w / (od + 1e-8)
                pk = lax.shift_left(col, 16) | row
                b1[0, pl.ds(j * 16, 16)] = pk
                b2[0, pl.ds(j * 16, 16)] = wn
            pltpu.sync_copy(b1.at[0], pk_hbm.at[pl.ds(base, C)])
            pltpu.sync_copy(b2.at[0], wn_hbm.at[pl.ds(base, C)])

        inv = jnp.full((16,), 1.0 / N, jnp.float32)

        @plsc.parallel_loop(0, NPAD // 16, unroll=8)
        def _(j):
            pr_v[pl.ds(j * 16, 16)] = inv

        def start_chunk0():
            pltpu.async_copy(pk_hbm.at[pl.ds(ebase, C)], b0.at[0], sem0)
            pltpu.async_copy(wn_hbm.at[pl.ds(ebase, C)], b2.at[0], sem1)

        def wait_chunk0():
            pltpu.make_async_copy(pk_hbm.at[pl.ds(ebase, C)], b0.at[0], sem0).wait()
            pltpu.make_async_copy(wn_hbm.at[pl.ds(ebase, C)], b2.at[0], sem1).wait()

        pltpu.sync_copy(pr_v.at[pl.ds(sbase, S)], prb_hbm.at[pl.ds(sbase, S)])
        full_barrier()
        pltpu.async_copy(prb_hbm, pr_v, sem2)
        zero_vmem(acc_v)
        start_chunk0()

        def iteration(i, _):
            pltpu.make_async_copy(prb_hbm, pr_v, sem2).wait()
            for c in range(NCH):
                ph = c & 1
                if c == 0:
                    wait_chunk0()
                if c + 1 < NCH:
                    nb = ebase + (c + 1) * C
                    nph = (c + 1) & 1
                    d_p = pltpu.async_copy(pk_hbm.at[pl.ds(nb, C)], b0.at[nph], sem0)
                    d_n = pltpu.async_copy(wn_hbm.at[pl.ds(nb, C)], b2.at[nph], sem1)

                @plsc.parallel_loop(0, NV, unroll=8)
                def _(j, ph=ph):
                    pk = b0[ph, pl.ds(j * 16, 16)]
                    wn = b2[ph, pl.ds(j * 16, 16)]
                    row = pk & 0xFFFF
                    col = lax.shift_right_logical(pk, 16)
                    prg = plsc.load_gather(pr_v, [row])
                    plsc.addupdate_scatter(acc_v, [col], prg * wn)
                if c + 1 < NCH:
                    d_p.wait()
                    d_n.wait()

            publish_partials(acc_v)
            full_barrier()
            stage_partials()
            reduce_slices_from_acc(damped=True)
            pltpu.sync_copy(acc_v.at[pl.ds(0, S)], prb_hbm.at[pl.ds(sbase, S)])
            zero_vmem(acc_v)
            full_barrier()
            pltpu.async_copy(prb_hbm, pr_v, sem2)
            start_chunk0()
            return 0

        lax.fori_loop(0, MAXIT, iteration, 0)

        pltpu.make_async_copy(prb_hbm, pr_v, sem2).wait()
        wait_chunk0()

        pltpu.sync_copy(pr_v.at[pl.ds(sbase, S)], pr_out.at[pl.ds(sbase, S)])

    return k


_sc_kernel = _make_kernel()


def kernel(edge_index, num_nodes, edge_weight):
    row = edge_index[0]
    col = edge_index[1]
    deg, pr = _sc_kernel(row, col, edge_weight)
    embedding_dim = 16
    pad = jnp.zeros((N, embedding_dim - 2), dtype=jnp.float32)
    return jnp.concatenate([deg[:N, None], pr[:N, None], pad], axis=1)
